# baseline (speedup 1.0000x reference)
import jax
import jax.numpy as jnp
from jax.experimental import pallas as pl
from jax.experimental.pallas import tpu as pltpu
from jax.experimental.pallas import tpu_sc as plsc

_BR = 128
_BC = 128
_L_MAIN = 99968


def _make_sc_kernel(B, L):
    mesh = plsc.VectorSubcoreMesh(core_axis_name="c", subcore_axis_name="s")

    @pl.kernel(
        out_type=jax.ShapeDtypeStruct((B, L), jnp.float32),
        mesh=mesh,
    )
    def sc_bias_add(x_hbm, b_hbm, o_hbm):
        def body(x_v, b_v, o_v):
            @pl.loop(0, _BC, step=16)
            def _(c):
                bvec = b_v.at[0, pl.ds(c, 16)][...]

                @pl.loop(0, _BR)
                def _(r):
                    o_v.at[r, pl.ds(c, 16)][...] = x_v.at[r, pl.ds(c, 16)][...] + bvec

        pltpu.emit_pipeline(
            body,
            grid=(B // _BR, _L_MAIN // _BC),
            in_specs=[
                pl.BlockSpec((_BR, _BC), lambda i, j: (i, j)),
                pl.BlockSpec((1, _BC), lambda i, j: (0, j)),
            ],
            out_specs=[pl.BlockSpec((_BR, _BC), lambda i, j: (i, j))],
            core_axis_name=("c", "s"),
            dimension_semantics=(pltpu.PARALLEL, pltpu.PARALLEL),
        )(x_hbm, b_hbm, o_hbm)

    return sc_bias_add


def _tail_kernel(xt_ref, bt_ref, o_ref):
    o_ref[...] = xt_ref[...] + bt_ref[...]


def _tail_add(x_tail, b_tail, B, tail):
    return pl.pallas_call(
        _tail_kernel,
        grid=(1,),
        in_specs=[
            pl.BlockSpec((B, tail), lambda i: (0, 0)),
            pl.BlockSpec((1, tail), lambda i: (0, 0)),
        ],
        out_specs=pl.BlockSpec((B, tail), lambda i: (0, 0)),
        out_shape=jax.ShapeDtypeStruct((B, tail), jnp.float32),
    )(x_tail, b_tail)


def kernel(user_emb, loc_logits, user_loc_weights, loc_bias):
    B, L = loc_logits.shape
    bias2d = loc_bias.reshape(1, L)
    out_main = _make_sc_kernel(B, L)(loc_logits, bias2d)
    tail_done = _tail_add(loc_logits[:, _L_MAIN:], bias2d[:, _L_MAIN:], B, L - _L_MAIN)
    return jax.lax.dynamic_update_slice(out_main, tail_done, (0, _L_MAIN))


# speedup vs baseline: 1.1360x; 1.1360x over previous
import jax
import jax.numpy as jnp
from jax.experimental import pallas as pl
from jax.experimental.pallas import tpu as pltpu
from jax.experimental.pallas import tpu_sc as plsc

_BR = 128
_BC = 128
_L_MAIN = 99968


def _make_sc_kernel(B, L):
    mesh = plsc.VectorSubcoreMesh(core_axis_name="c", subcore_axis_name="s")

    @pl.kernel(
        out_type=jax.ShapeDtypeStruct((B, L), jnp.float32),
        mesh=mesh,
    )
    def sc_bias_add(x_hbm, b_hbm, o_hbm):
        def body(x_v, b_v, o_v):
            @pl.loop(0, _BC, step=16)
            def _(c):
                bvec = b_v.at[0, pl.ds(c, 16)][...]

                @pl.loop(0, _BR, unroll=16)
                def _(r):
                    o_v.at[r, pl.ds(c, 16)][...] = x_v.at[r, pl.ds(c, 16)][...] + bvec

        pltpu.emit_pipeline(
            body,
            grid=(B // _BR, _L_MAIN // _BC),
            in_specs=[
                pl.BlockSpec((_BR, _BC), lambda i, j: (i, j)),
                pl.BlockSpec((1, _BC), lambda i, j: (0, j)),
            ],
            out_specs=[pl.BlockSpec((_BR, _BC), lambda i, j: (i, j))],
            core_axis_name=("c", "s"),
            dimension_semantics=(pltpu.PARALLEL, pltpu.PARALLEL),
        )(x_hbm, b_hbm, o_hbm)

    return sc_bias_add


def _tail_kernel(xt_ref, bt_ref, o_ref):
    o_ref[...] = xt_ref[...] + bt_ref[...]


def _tail_add(x_tail, b_tail, B, tail):
    return pl.pallas_call(
        _tail_kernel,
        grid=(1,),
        in_specs=[
            pl.BlockSpec((B, tail), lambda i: (0, 0)),
            pl.BlockSpec((1, tail), lambda i: (0, 0)),
        ],
        out_specs=pl.BlockSpec((B, tail), lambda i: (0, 0)),
        out_shape=jax.ShapeDtypeStruct((B, tail), jnp.float32),
    )(x_tail, b_tail)


def kernel(user_emb, loc_logits, user_loc_weights, loc_bias):
    B, L = loc_logits.shape
    bias2d = loc_bias.reshape(1, L)
    out_main = _make_sc_kernel(B, L)(loc_logits, bias2d)
    tail_done = _tail_add(loc_logits[:, _L_MAIN:], bias2d[:, _L_MAIN:], B, L - _L_MAIN)
    return jax.lax.dynamic_update_slice(out_main, tail_done, (0, _L_MAIN))


# speedup vs baseline: 1.8971x; 1.6700x over previous
import jax
import jax.numpy as jnp
from jax.experimental import pallas as pl
from jax.experimental.pallas import tpu as pltpu

_CR = 8
_NBUF = 8
_CT = 1024


def _bias_add_kernel(x_hbm, b_vmem, o_hbm, *scratch):
    n_chunks = x_hbm.shape[0] // _CR
    L = x_hbm.shape[1]
    in_bufs = scratch[0:_NBUF]
    out_bufs = scratch[_NBUF:2 * _NBUF]
    in_sems = scratch[2 * _NBUF:3 * _NBUF]
    out_sems = scratch[3 * _NBUF:4 * _NBUF]

    def in_copy(chunk, b):
        return pltpu.make_async_copy(
            x_hbm.at[pl.ds(chunk * _CR, _CR), :], in_bufs[b], in_sems[b])

    def out_copy(chunk, b):
        return pltpu.make_async_copy(
            out_bufs[b], o_hbm.at[pl.ds(chunk * _CR, _CR), :], out_sems[b])

    for s in range(_NBUF):
        in_copy(s, s).start()

    for i in range(n_chunks):
        b = i % _NBUF
        if i >= _NBUF:
            out_copy(i - _NBUF, b).wait()
        in_copy(i, b).wait()

        def compute(c, _):
            sl = pl.ds(c * _CT, _CT)
            out_bufs[b][:, sl] = in_bufs[b][:, sl] + b_vmem[:, sl]
            return 0

        n_full = L // _CT
        jax.lax.fori_loop(0, n_full, compute, 0, unroll=2)
        if L % _CT:
            rem = slice(n_full * _CT, L)
            out_bufs[b][:, rem] = in_bufs[b][:, rem] + b_vmem[:, rem]

        out_copy(i, b).start()
        if i + _NBUF < n_chunks:
            in_copy(i + _NBUF, b).start()

    for i in range(n_chunks - _NBUF, n_chunks):
        out_copy(i, i % _NBUF).wait()


def kernel(user_emb, loc_logits, user_loc_weights, loc_bias):
    B, L = loc_logits.shape
    bias2d = loc_bias.reshape(1, L)
    vbuf = lambda: pltpu.VMEM((_CR, L), jnp.float32)
    out = pl.pallas_call(
        _bias_add_kernel,
        in_specs=[
            pl.BlockSpec(memory_space=pltpu.MemorySpace.HBM),
            pl.BlockSpec(memory_space=pltpu.VMEM),
        ],
        out_specs=pl.BlockSpec(memory_space=pltpu.MemorySpace.HBM),
        out_shape=jax.ShapeDtypeStruct((B, L), jnp.float32),
        scratch_shapes=(
            [vbuf() for _ in range(_NBUF)]
            + [vbuf() for _ in range(_NBUF)]
            + [pltpu.SemaphoreType.DMA for _ in range(2 * _NBUF)]
        ),
        compiler_params=pltpu.CompilerParams(vmem_limit_bytes=60 * 1024 * 1024),
    )(loc_logits, bias2d)
    return out
